# Initial kernel scaffold; baseline (speedup 1.0000x reference)
import jax
import jax.numpy as jnp
from jax.experimental import pallas as pl


def kernel(atom_layer_input, bond_layer_input, atom_adjacency_graph, atom_bond_adjacency_graph, bond_atom_adjacency_graph, U_w, U_b, V_w, V_b, A_w, A_b, B_w, B_b, C_w, C_b, bn_bond_gamma, bn_bond_beta, bn_atom_gamma, bn_atom_beta):
    raise NotImplementedError("write your pallas kernel here")


# speedup vs baseline: 3.1617x; 3.1617x over previous
import functools

import jax
import jax.numpy as jnp
from jax import lax
from jax.experimental import pallas as pl
from jax.experimental.pallas import tpu as pltpu
from jax.experimental.pallas import tpu_sc as plsc

H = 128
L = 16
NC = 2
NS = 16
NW = NC * NS
CH_E = 96
CH_A = 40
DEG = 6


def _sc_mesh():
    return plsc.VectorSubcoreMesh(
        core_axis_name="c", subcore_axis_name="s", num_cores=NC, num_subcores=NS
    )


def _atom_proj_body(x_ref, w_ref, b_ref, u_ref, v_ref, bb_ref, cc_ref):
    y = jnp.dot(x_ref[...], w_ref[...], preferred_element_type=jnp.float32)
    y = y + b_ref[...]
    u_ref[...] = y[:, 0 * H:1 * H]
    v_ref[...] = y[:, 1 * H:2 * H]
    bb_ref[...] = y[:, 2 * H:3 * H]
    cc_ref[...] = y[:, 3 * H:4 * H]


def _bond_proj_body(x_ref, w_ref, b_ref, o_ref):
    o_ref[...] = (
        jnp.dot(x_ref[...], w_ref[...], preferred_element_type=jnp.float32)
        + b_ref[...]
    )


def _edge_body(E_total, atomB, atomC, bondA, src, dst,
               egsi, gates, stats,
               src_v, dst_v, rowsB, rowsC, ba_v, eg_v, gt_v, st_v, sem):
    c = lax.axis_index("c")
    s = lax.axis_index("s")
    wid = s * NC + c
    tch = E_total // CH_E
    n_t = (tch - 1 - wid) // NW + 1

    def chunk(t, accs):
        cid = wid + t * NW
        base = cid * CH_E
        pltpu.sync_copy(src.at[pl.ds(base, CH_E)], src_v)
        pltpu.sync_copy(dst.at[pl.ds(base, CH_E)], dst_v)
        cp1 = pltpu.async_copy(atomB.at[src_v], rowsB, sem)
        cp2 = pltpu.async_copy(atomC.at[dst_v], rowsC, sem)
        pltpu.sync_copy(bondA.at[pl.ds(base, CH_E)], ba_v)
        cp1.wait()
        cp2.wait()

        def row(r, acc):
            acc_s, acc_q = acc
            new_s, new_q = [], []
            for h in range(H // L):
                sl = pl.ds(h * L, L)
                x = rowsB[r, sl] + rowsC[r, sl] + ba_v[r, sl]
                g = 1.0 / (1.0 + jnp.exp(-x))
                eg_v[r, sl] = x
                gt_v[r, sl] = g
                new_s.append(acc_s[h] + x)
                new_q.append(acc_q[h] + x * x)
            return (tuple(new_s), tuple(new_q))

        accs = lax.fori_loop(0, CH_E, row, accs)
        pltpu.sync_copy(eg_v, egsi.at[pl.ds(base, CH_E)])
        pltpu.sync_copy(gt_v, gates.at[pl.ds(base, CH_E)])
        return accs

    zero = jnp.zeros((L,), jnp.float32)
    init = (tuple(zero for _ in range(H // L)), tuple(zero for _ in range(H // L)))
    acc_s, acc_q = lax.fori_loop(0, n_t, chunk, init)
    for h in range(H // L):
        st_v[0, pl.ds(h * L, L)] = acc_s[h]
        st_v[1, pl.ds(h * L, L)] = acc_q[h]
    pltpu.sync_copy(st_v, stats.at[wid])


def _atom_body(N_total, atomU, atomV, gates, aadjT, abndT,
               afsi, stats,
               ai_v, bi_v, vr_v, gr_v, u_v, st_v, sem):
    c = lax.axis_index("c")
    s = lax.axis_index("s")
    wid = s * NC + c
    tch = N_total // CH_A
    n_t = (tch - 1 - wid) // NW + 1

    def chunk(t, accs):
        cid = wid + t * NW
        base = cid * CH_A
        for k in range(DEG):
            pltpu.sync_copy(aadjT.at[pl.ds(k * N_total + base, CH_A)], ai_v.at[k])
            pltpu.sync_copy(abndT.at[pl.ds(k * N_total + base, CH_A)], bi_v.at[k])
        cps = []
        for k in range(DEG):
            cps.append(pltpu.async_copy(atomV.at[ai_v.at[k]], vr_v.at[k], sem))
            cps.append(pltpu.async_copy(gates.at[bi_v.at[k]], gr_v.at[k], sem))
        pltpu.sync_copy(atomU.at[pl.ds(base, CH_A)], u_v)
        for cp in cps:
            cp.wait()

        def row(r, acc):
            acc_s, acc_q = acc
            new_s, new_q = [], []
            for h in range(H // L):
                sl = pl.ds(h * L, L)
                x = u_v[r, sl]
                for k in range(DEG):
                    x = x + vr_v[k, r, sl] * gr_v[k, r, sl]
                u_v[r, sl] = x
                new_s.append(acc_s[h] + x)
                new_q.append(acc_q[h] + x * x)
            return (tuple(new_s), tuple(new_q))

        accs = lax.fori_loop(0, CH_A, row, accs)
        pltpu.sync_copy(u_v, afsi.at[pl.ds(base, CH_A)])
        return accs

    zero = jnp.zeros((L,), jnp.float32)
    init = (tuple(zero for _ in range(H // L)), tuple(zero for _ in range(H // L)))
    acc_s, acc_q = lax.fori_loop(0, n_t, chunk, init)
    for h in range(H // L):
        st_v[0, pl.ds(h * L, L)] = acc_s[h]
        st_v[1, pl.ds(h * L, L)] = acc_q[h]
    pltpu.sync_copy(st_v, stats.at[wid])


def _norm_stats(stats, count):
    tot = jnp.sum(stats[:, 0, :], axis=0)
    totq = jnp.sum(stats[:, 1, :], axis=0)
    mean = tot / count
    var = totq / count - mean * mean
    rstd = lax.rsqrt(var + 1e-5)
    return mean, rstd


def _bond_norm_body(count, x_ref, stats_ref, g_ref, b_ref, o_ref):
    mean, rstd = _norm_stats(stats_ref[...], count)
    scale = (rstd * g_ref[0])[None, :]
    shift = (b_ref[0] - mean * rstd * g_ref[0])[None, :]
    o_ref[...] = jnp.maximum(x_ref[...] * scale + shift, 0.0)


def _atom_norm_body(count, x_ref, res_ref, stats_ref, g_ref, b_ref, o_ref):
    mean, rstd = _norm_stats(stats_ref[...], count)
    scale = (rstd * g_ref[0])[None, :]
    shift = (b_ref[0] - mean * rstd * g_ref[0])[None, :]
    o_ref[...] = jnp.maximum(x_ref[...] * scale + shift, 0.0) + res_ref[...]


def kernel(atom_layer_input, bond_layer_input, atom_adjacency_graph,
           atom_bond_adjacency_graph, bond_atom_adjacency_graph,
           U_w, U_b, V_w, V_b, A_w, A_b, B_w, B_b, C_w, C_b,
           bn_bond_gamma, bn_bond_beta, bn_atom_gamma, bn_atom_beta):
    N = atom_layer_input.shape[0]
    E = bond_layer_input.shape[0]
    f32 = jnp.float32

    src = bond_atom_adjacency_graph[:, 0].astype(jnp.int32)
    dst = bond_atom_adjacency_graph[:, 1].astype(jnp.int32)
    aadjT = atom_adjacency_graph.astype(jnp.int32).T.reshape(-1)
    abndT = atom_bond_adjacency_graph.astype(jnp.int32).T.reshape(-1)

    w_cat = jnp.concatenate([U_w.T, V_w.T, B_w.T, C_w.T], axis=1)
    b_cat = jnp.concatenate(
        [U_b, V_b, jnp.zeros_like(B_b), jnp.zeros_like(C_b)])[None, :]

    bm_a = 1000
    atomU, atomV, atomB, atomC = pl.pallas_call(
        _atom_proj_body,
        grid=(N // bm_a,),
        in_specs=[
            pl.BlockSpec((bm_a, H), lambda i: (i, 0)),
            pl.BlockSpec((H, 4 * H), lambda i: (0, 0)),
            pl.BlockSpec((1, 4 * H), lambda i: (0, 0)),
        ],
        out_specs=[pl.BlockSpec((bm_a, H), lambda i: (i, 0))] * 4,
        out_shape=[jax.ShapeDtypeStruct((N, H), f32)] * 4,
    )(atom_layer_input, w_cat, b_cat)

    bm_b = 3000
    bondA = pl.pallas_call(
        _bond_proj_body,
        grid=(E // bm_b,),
        in_specs=[
            pl.BlockSpec((bm_b, H), lambda i: (i, 0)),
            pl.BlockSpec((H, H), lambda i: (0, 0)),
            pl.BlockSpec((1, H), lambda i: (0, 0)),
        ],
        out_specs=pl.BlockSpec((bm_b, H), lambda i: (i, 0)),
        out_shape=jax.ShapeDtypeStruct((E, H), f32),
    )(bond_layer_input, A_w.T, (A_b + B_b + C_b)[None, :])

    edge_call = pl.kernel(
        functools.partial(_edge_body, E),
        out_type=(
            jax.ShapeDtypeStruct((E, H), f32),
            jax.ShapeDtypeStruct((E, H), f32),
            jax.ShapeDtypeStruct((NW, 2, H), f32),
        ),
        mesh=_sc_mesh(),
        scratch_types=[
            pltpu.VMEM((CH_E,), jnp.int32),
            pltpu.VMEM((CH_E,), jnp.int32),
            pltpu.VMEM((CH_E, H), f32),
            pltpu.VMEM((CH_E, H), f32),
            pltpu.VMEM((CH_E, H), f32),
            pltpu.VMEM((CH_E, H), f32),
            pltpu.VMEM((CH_E, H), f32),
            pltpu.VMEM((2, H), f32),
            pltpu.SemaphoreType.DMA,
        ],
    )
    egsi, gates, stats_b = edge_call(atomB, atomC, bondA, src, dst)

    bond_layer_output = pl.pallas_call(
        functools.partial(_bond_norm_body, float(E)),
        grid=(E // bm_b,),
        in_specs=[
            pl.BlockSpec((bm_b, H), lambda i: (i, 0)),
            pl.BlockSpec((NW, 2, H), lambda i: (0, 0, 0)),
            pl.BlockSpec((1, H), lambda i: (0, 0)),
            pl.BlockSpec((1, H), lambda i: (0, 0)),
        ],
        out_specs=pl.BlockSpec((bm_b, H), lambda i: (i, 0)),
        out_shape=jax.ShapeDtypeStruct((E, H), f32),
    )(egsi, stats_b, bn_bond_gamma[None, :], bn_bond_beta[None, :])

    atom_call = pl.kernel(
        functools.partial(_atom_body, N),
        out_type=(
            jax.ShapeDtypeStruct((N, H), f32),
            jax.ShapeDtypeStruct((NW, 2, H), f32),
        ),
        mesh=_sc_mesh(),
        scratch_types=[
            pltpu.VMEM((DEG, CH_A), jnp.int32),
            pltpu.VMEM((DEG, CH_A), jnp.int32),
            pltpu.VMEM((DEG, CH_A, H), f32),
            pltpu.VMEM((DEG, CH_A, H), f32),
            pltpu.VMEM((CH_A, H), f32),
            pltpu.VMEM((2, H), f32),
            pltpu.SemaphoreType.DMA,
        ],
    )
    afsi, stats_a = atom_call(atomU, atomV, gates, aadjT, abndT)

    atom_layer_output = pl.pallas_call(
        functools.partial(_atom_norm_body, float(N)),
        grid=(N // bm_a,),
        in_specs=[
            pl.BlockSpec((bm_a, H), lambda i: (i, 0)),
            pl.BlockSpec((bm_a, H), lambda i: (i, 0)),
            pl.BlockSpec((NW, 2, H), lambda i: (0, 0, 0)),
            pl.BlockSpec((1, H), lambda i: (0, 0)),
            pl.BlockSpec((1, H), lambda i: (0, 0)),
        ],
        out_specs=pl.BlockSpec((bm_a, H), lambda i: (i, 0)),
        out_shape=jax.ShapeDtypeStruct((N, H), f32),
    )(afsi, atom_layer_input, stats_a, bn_atom_gamma[None, :],
      bn_atom_beta[None, :])

    return (atom_layer_output, bond_layer_output)
